# baseline (speedup 1.0000x reference)
import numpy as np
import jax
import jax.numpy as jnp
from jax import lax
from jax.experimental import pallas as pl
from jax.experimental.pallas import tpu as pltpu

_DN = (((1,), (1,)), ((), ()))


def _attn_body(ids_ref, x_ref, wq_ref, wk_ref, wv_ref, b_ref, nrm_ref,
               avg_ref, std_ref, out_ref):
    S = x_ref.shape[1]
    P = S - 1
    D = x_ref.shape[2]
    gd = float(np.sqrt(P))

    xx = x_ref[0]
    q = lax.dot_general(xx, wq_ref[...], _DN,
                        preferred_element_type=jnp.float32) + b_ref[0:1, :]
    k = lax.dot_general(xx, wk_ref[...], _DN,
                        preferred_element_type=jnp.float32) + b_ref[1:2, :]
    v = lax.dot_general(xx, wv_ref[...], _DN,
                        preferred_element_type=jnp.float32) + b_ref[2:3, :]

    A_T = lax.dot_general(k, q[1:, :], _DN,
                          preferred_element_type=jnp.float32)

    nx = nrm_ref[0, 0:1, :]
    ny = nrm_ref[0, 1:2, :]
    key_x = (nx - avg_ref[0, 0:1, :]) / std_ref[0, 0:1, :]
    key_y = (ny - avg_ref[0, 1:2, :]) / std_ref[0, 1:2, :]
    kx1 = jnp.ceil(key_x)
    kx2 = jnp.floor(key_x)
    ky1 = jnp.ceil(key_y)
    ky2 = jnp.floor(key_y)
    idx = [jnp.mod((gd * ky + kx).astype(jnp.int32), S)
           for ky in (ky1, ky2) for kx in (kx1, kx2)]

    row = lax.broadcasted_iota(jnp.int32, (S, P), 0)
    sel = [row == idx[t] for t in range(4)]

    scores = jnp.concatenate(
        [jnp.sum(jnp.where(sel[t], A_T, 0.0), axis=0, keepdims=True)
         for t in range(4)], axis=0)
    m = jnp.max(scores, axis=0, keepdims=True)
    e = jnp.exp(scores - m)
    w = e / jnp.sum(e, axis=0, keepdims=True)

    M_T = jnp.zeros((S, P), jnp.float32)
    for t in range(4):
        M_T = M_T + jnp.where(sel[t], w[t:t + 1, :], 0.0)

    out1 = lax.dot_general(M_T, v, (((0,), (0,)), ((), ())),
                           preferred_element_type=jnp.float32)
    out_ref[0, 0:1, :] = jnp.ones((1, D), jnp.float32)
    out_ref[0, 1:, :] = out1


def kernel(x, img_ids, mask, W_q, b_q, W_k, b_k, W_v, b_v, avgs, std_devs):
    B, S, D = x.shape
    P = S - 1

    base = jax.random.key(42)
    rows = []
    for j in range(B):
        nx = jax.random.normal(jax.random.fold_in(base, 2 * j), (1, P),
                               dtype=jnp.float32)
        ny = jax.random.normal(jax.random.fold_in(base, 2 * j + 1), (1, P),
                               dtype=jnp.float32)
        rows.append(jnp.concatenate([nx, ny], axis=0))
    norms = jnp.stack(rows)

    b_all = jnp.stack([b_q, b_k, b_v])

    grid_spec = pltpu.PrefetchScalarGridSpec(
        num_scalar_prefetch=1,
        grid=(B,),
        in_specs=[
            pl.BlockSpec((1, S, D), lambda j, ids: (j, 0, 0)),
            pl.BlockSpec((D, D), lambda j, ids: (0, 0)),
            pl.BlockSpec((D, D), lambda j, ids: (0, 0)),
            pl.BlockSpec((D, D), lambda j, ids: (0, 0)),
            pl.BlockSpec((3, D), lambda j, ids: (0, 0)),
            pl.BlockSpec((1, 2, P), lambda j, ids: (j, 0, 0)),
            pl.BlockSpec((1, 2, P), lambda j, ids: (ids[j], 0, 0)),
            pl.BlockSpec((1, 2, P), lambda j, ids: (ids[j], 0, 0)),
        ],
        out_specs=pl.BlockSpec((1, S, D), lambda j, ids: (j, 0, 0)),
    )
    out = pl.pallas_call(
        _attn_body,
        grid_spec=grid_spec,
        out_shape=jax.ShapeDtypeStruct((B, S, D), jnp.float32),
        compiler_params=pltpu.CompilerParams(
            dimension_semantics=("parallel",)),
    )(img_ids, x, W_q, W_k, W_v, b_all, norms, avgs, std_devs)
    return out


# speedup vs baseline: 1.0094x; 1.0094x over previous
import numpy as np
import jax
import jax.numpy as jnp
from jax import lax
from jax.experimental import pallas as pl
from jax.experimental.pallas import tpu as pltpu

_DN = (((1,), (1,)), ((), ()))


def _attn_body(ids_ref, x_ref, wq_ref, wk_ref, wv_ref, b_ref, nrm_ref,
               avg_ref, std_ref, out_ref):
    S = x_ref.shape[1]
    P = S - 1
    D = x_ref.shape[2]
    gd = float(np.sqrt(P))

    xx = x_ref[0]
    q = lax.dot_general(xx, wq_ref[...], _DN,
                        preferred_element_type=jnp.float32) + b_ref[0:1, :]
    k = lax.dot_general(xx, wk_ref[...], _DN,
                        preferred_element_type=jnp.float32) + b_ref[1:2, :]
    v = lax.dot_general(xx.astype(jnp.bfloat16),
                        wv_ref[...].astype(jnp.bfloat16), _DN,
                        preferred_element_type=jnp.float32) + b_ref[2:3, :]

    A_T = lax.dot_general(k, q[1:, :], _DN,
                          preferred_element_type=jnp.float32)

    nx = nrm_ref[0, 0:1, :]
    ny = nrm_ref[0, 1:2, :]
    key_x = (nx - avg_ref[0, 0:1, :]) / std_ref[0, 0:1, :]
    key_y = (ny - avg_ref[0, 1:2, :]) / std_ref[0, 1:2, :]
    kx1 = jnp.ceil(key_x)
    kx2 = jnp.floor(key_x)
    ky1 = jnp.ceil(key_y)
    ky2 = jnp.floor(key_y)
    idx = [jnp.mod((gd * ky + kx).astype(jnp.int32), S)
           for ky in (ky1, ky2) for kx in (kx1, kx2)]

    row = lax.broadcasted_iota(jnp.int32, (S, P), 0)
    sel = [row == idx[t] for t in range(4)]

    scores = jnp.concatenate(
        [jnp.sum(jnp.where(sel[t], A_T, 0.0), axis=0, keepdims=True)
         for t in range(4)], axis=0)
    m = jnp.max(scores, axis=0, keepdims=True)
    e = jnp.exp(scores - m)
    w = e / jnp.sum(e, axis=0, keepdims=True)

    M_T = jnp.zeros((S, P), jnp.float32)
    for t in range(4):
        M_T = M_T + jnp.where(sel[t], w[t:t + 1, :], 0.0)

    out1 = lax.dot_general(M_T.astype(jnp.bfloat16), v.astype(jnp.bfloat16),
                           (((0,), (0,)), ((), ())),
                           preferred_element_type=jnp.float32)
    out_ref[0, 0:1, :] = jnp.ones((1, D), jnp.float32)
    out_ref[0, 1:, :] = out1


def kernel(x, img_ids, mask, W_q, b_q, W_k, b_k, W_v, b_v, avgs, std_devs):
    B, S, D = x.shape
    P = S - 1

    base = jax.random.key(42)
    rows = []
    for j in range(B):
        nx = jax.random.normal(jax.random.fold_in(base, 2 * j), (1, P),
                               dtype=jnp.float32)
        ny = jax.random.normal(jax.random.fold_in(base, 2 * j + 1), (1, P),
                               dtype=jnp.float32)
        rows.append(jnp.concatenate([nx, ny], axis=0))
    norms = jnp.stack(rows)

    b_all = jnp.stack([b_q, b_k, b_v])

    grid_spec = pltpu.PrefetchScalarGridSpec(
        num_scalar_prefetch=1,
        grid=(B,),
        in_specs=[
            pl.BlockSpec((1, S, D), lambda j, ids: (j, 0, 0)),
            pl.BlockSpec((D, D), lambda j, ids: (0, 0)),
            pl.BlockSpec((D, D), lambda j, ids: (0, 0)),
            pl.BlockSpec((D, D), lambda j, ids: (0, 0)),
            pl.BlockSpec((3, D), lambda j, ids: (0, 0)),
            pl.BlockSpec((1, 2, P), lambda j, ids: (j, 0, 0)),
            pl.BlockSpec((1, 2, P), lambda j, ids: (ids[j], 0, 0)),
            pl.BlockSpec((1, 2, P), lambda j, ids: (ids[j], 0, 0)),
        ],
        out_specs=pl.BlockSpec((1, S, D), lambda j, ids: (j, 0, 0)),
    )
    out = pl.pallas_call(
        _attn_body,
        grid_spec=grid_spec,
        out_shape=jax.ShapeDtypeStruct((B, S, D), jnp.float32),
        compiler_params=pltpu.CompilerParams(
            dimension_semantics=("parallel",)),
    )(img_ids, x, W_q, W_k, W_v, b_all, norms, avgs, std_devs)
    return out
